# baseline (speedup 1.0000x reference)
import functools

import jax
import jax.numpy as jnp
from jax import lax
from jax.experimental import pallas as pl
from jax.experimental.pallas import tpu as pltpu
from jax.experimental.pallas import tpu_sc as plsc

M = 16384
N = 16384
B = 64

NC = 2
NS = 16
NW = NC * NS
L = 16

CH = 4096
SLOTS = 4
GRP = CH // L
GI = 8


def _sc_spmm(nchunks):
    mesh = plsc.VectorSubcoreMesh(core_axis_name="c", subcore_axis_name="s")

    @functools.partial(
        pl.kernel,
        out_type=jax.ShapeDtypeStruct((B, M), jnp.float32),
        mesh=mesh,
        scratch_types=[
            pltpu.VMEM((N,), jnp.int32),
            pltpu.VMEM((M,), jnp.float32),
            pltpu.VMEM((M,), jnp.float32),
            pltpu.VMEM((SLOTS, CH), jnp.int32),
            pltpu.VMEM((SLOTS, CH), jnp.float32),
            pltpu.SemaphoreType.DMA,
            pltpu.SemaphoreType.DMA,
        ],
        compiler_params=pltpu.CompilerParams(needs_layout_passes=False),
    )
    def kfn(xpack, pidx, vals, bias, out, xab, ya, yb, iv, vv, semx, semd):
        wid = lax.axis_index("s") * NC + lax.axis_index("c")

        pltpu.async_copy(xpack.at[wid], xab, semx)
        pltpu.async_copy(bias, ya, semx)
        pltpu.async_copy(bias, yb, semx)

        def start(c, slot):
            pltpu.make_async_copy(
                pidx.at[pl.ds(c * CH, CH)], iv.at[slot], semd).start()
            pltpu.make_async_copy(
                vals.at[pl.ds(c * CH, CH)], vv.at[slot], semd).start()

        def wait(c, slot):
            pltpu.make_async_copy(
                pidx.at[pl.ds(c * CH, CH)], iv.at[slot], semd).wait()
            pltpu.make_async_copy(
                vals.at[pl.ds(c * CH, CH)], vv.at[slot], semd).wait()

        for s in range(SLOTS):
            start(s, s)
        pltpu.make_async_copy(xpack.at[wid], xab, semx).wait()
        pltpu.make_async_copy(bias, ya, semx).wait()
        pltpu.make_async_copy(bias, yb, semx).wait()

        def chunk_body(c, slot):
            wait(c, slot)

            def groups(g, _):
                base = g * (L * GI)
                pvs = [iv[slot, pl.ds(base + k * L, L)] for k in range(GI)]
                vs = [vv[slot, pl.ds(base + k * L, L)] for k in range(GI)]
                col = [pv & 0xFFFF for pv in pvs]
                row = [lax.shift_right_logical(pv, 16) for pv in pvs]
                pk = [plsc.load_gather(xab, [c2]) for c2 in col]
                ab = [
                    plsc.unpack(
                        plsc.bitcast(p, jnp.bfloat16),
                        format=plsc.PackFormat.INTERLEAVED,
                        preferred_element_type=jnp.float32,
                    )
                    for p in pk
                ]
                for k in range(GI):
                    plsc.addupdate_scatter(ya, [row[k]], ab[k][0] * vs[k])
                    plsc.addupdate_scatter(yb, [row[k]], ab[k][1] * vs[k])
                return 0

            lax.fori_loop(0, GRP // GI, groups, 0)

        def outer(i, _):
            for s in range(SLOTS):
                c = i * SLOTS + s
                chunk_body(c, s)
                @pl.when(c + SLOTS < nchunks)
                def _():
                    start(c + SLOTS, s)
            return 0

        lax.fori_loop(0, nchunks // SLOTS, outer, 0)

        pltpu.sync_copy(ya, out.at[wid])
        pltpu.sync_copy(yb, out.at[wid + NW])

    return kfn


def kernel(x, sparse_weight_indices, sparse_weight_values, bias):
    rows = sparse_weight_indices[0]
    cols = sparse_weight_indices[1]
    nnz = rows.shape[0]

    nnzp = ((nnz + SLOTS * CH - 1) // (SLOTS * CH)) * (SLOTS * CH)
    pad = nnzp - nnz
    pidx = (rows.astype(jnp.int32) << 16) | cols.astype(jnp.int32)
    pidx = jnp.pad(pidx, (0, pad))
    vals = jnp.pad(sparse_weight_values, (0, pad))

    lo = lax.bitcast_convert_type(
        x[:NW].astype(jnp.bfloat16), jnp.uint16).astype(jnp.uint32)
    hi = lax.bitcast_convert_type(
        x[NW:].astype(jnp.bfloat16), jnp.uint16).astype(jnp.uint32)
    xpack = lax.bitcast_convert_type((hi << 16) | lo, jnp.int32)

    return _sc_spmm(nnzp // CH)(xpack, pidx, vals, bias)


# speedup vs baseline: 1.0805x; 1.0805x over previous
import functools

import jax
import jax.numpy as jnp
from jax import lax
from jax.experimental import pallas as pl
from jax.experimental.pallas import tpu as pltpu
from jax.experimental.pallas import tpu_sc as plsc

M = 16384
N = 16384
B = 64

NC = 2
NS = 16
NW = NC * NS
L = 16

CH = 2048
SLOTS = 8
GRP = CH // L
GI = 8


def _sc_spmm(nchunks):
    mesh = plsc.VectorSubcoreMesh(core_axis_name="c", subcore_axis_name="s")

    @functools.partial(
        pl.kernel,
        out_type=jax.ShapeDtypeStruct((B, M), jnp.float32),
        mesh=mesh,
        scratch_types=[
            pltpu.VMEM((N,), jnp.int32),
            pltpu.VMEM((M,), jnp.float32),
            pltpu.VMEM((M,), jnp.float32),
            pltpu.VMEM((SLOTS, CH), jnp.int32),
            pltpu.VMEM((SLOTS, CH), jnp.float32),
            pltpu.SemaphoreType.DMA,
            pltpu.SemaphoreType.DMA,
        ],
        compiler_params=pltpu.CompilerParams(needs_layout_passes=False),
    )
    def kfn(xpack, pidx, vals, bias, out, xab, ya, yb, iv, vv, semx, semd):
        wid = lax.axis_index("s") * NC + lax.axis_index("c")

        pltpu.async_copy(xpack.at[wid], xab, semx)
        pltpu.async_copy(bias, ya, semx)
        pltpu.async_copy(bias, yb, semx)

        def start(c, slot):
            pltpu.make_async_copy(
                pidx.at[pl.ds(c * CH, CH)], iv.at[slot], semd).start()
            pltpu.make_async_copy(
                vals.at[pl.ds(c * CH, CH)], vv.at[slot], semd).start()

        def wait(c, slot):
            pltpu.make_async_copy(
                pidx.at[pl.ds(c * CH, CH)], iv.at[slot], semd).wait()
            pltpu.make_async_copy(
                vals.at[pl.ds(c * CH, CH)], vv.at[slot], semd).wait()

        for s in range(SLOTS):
            start(s, s)
        pltpu.make_async_copy(xpack.at[wid], xab, semx).wait()
        pltpu.make_async_copy(bias, ya, semx).wait()
        pltpu.make_async_copy(bias, yb, semx).wait()

        def chunk_body(c, slot):
            wait(c, slot)

            def groups(g, _):
                base = g * (L * GI)
                pvs = [iv[slot, pl.ds(base + k * L, L)] for k in range(GI)]
                vs = [vv[slot, pl.ds(base + k * L, L)] for k in range(GI)]
                col = [pv & 0xFFFF for pv in pvs]
                row = [lax.shift_right_logical(pv, 16) for pv in pvs]
                pk = [plsc.load_gather(xab, [c2]) for c2 in col]
                ab = [
                    plsc.unpack(
                        plsc.bitcast(p, jnp.bfloat16),
                        format=plsc.PackFormat.INTERLEAVED,
                        preferred_element_type=jnp.float32,
                    )
                    for p in pk
                ]
                for k in range(GI):
                    plsc.addupdate_scatter(ya, [row[k]], ab[k][0] * vs[k])
                    plsc.addupdate_scatter(yb, [row[k]], ab[k][1] * vs[k])
                return 0

            lax.fori_loop(0, GRP // GI, groups, 0)

        def outer(i, _):
            for s in range(SLOTS):
                c = i * SLOTS + s
                chunk_body(c, s)
                @pl.when(c + SLOTS < nchunks)
                def _():
                    start(c + SLOTS, s)
            return 0

        lax.fori_loop(0, nchunks // SLOTS, outer, 0)

        pltpu.sync_copy(ya, out.at[wid])
        pltpu.sync_copy(yb, out.at[wid + NW])

    return kfn


def kernel(x, sparse_weight_indices, sparse_weight_values, bias):
    rows = sparse_weight_indices[0]
    cols = sparse_weight_indices[1]
    nnz = rows.shape[0]

    nnzp = ((nnz + SLOTS * CH - 1) // (SLOTS * CH)) * (SLOTS * CH)
    pad = nnzp - nnz
    pidx = (rows.astype(jnp.int32) << 16) | cols.astype(jnp.int32)
    pidx = jnp.pad(pidx, (0, pad))
    vals = jnp.pad(sparse_weight_values, (0, pad))

    lo = lax.bitcast_convert_type(
        x[:NW].astype(jnp.bfloat16), jnp.uint16).astype(jnp.uint32)
    hi = lax.bitcast_convert_type(
        x[NW:].astype(jnp.bfloat16), jnp.uint16).astype(jnp.uint32)
    xpack = lax.bitcast_convert_type((hi << 16) | lo, jnp.int32)

    return _sc_spmm(nnzp // CH)(xpack, pidx, vals, bias)


# speedup vs baseline: 1.2352x; 1.1432x over previous
import functools

import jax
import jax.numpy as jnp
from jax import lax
from jax.experimental import pallas as pl
from jax.experimental.pallas import tpu as pltpu
from jax.experimental.pallas import tpu_sc as plsc

M = 16384
N = 16384
B = 64

NC = 2
NS = 16
NW = NC * NS
L = 16

CH = 2048
SLOTS = 4
GRP = CH // L
GI = 8


def _sc_spmm(nchunks):
    mesh = plsc.VectorSubcoreMesh(core_axis_name="c", subcore_axis_name="s")

    @functools.partial(
        pl.kernel,
        out_type=jax.ShapeDtypeStruct((B, M), jnp.float32),
        mesh=mesh,
        scratch_types=[
            pltpu.VMEM((N,), jnp.int32),
            pltpu.VMEM((M,), jnp.float32),
            pltpu.VMEM((M,), jnp.float32),
            pltpu.VMEM((SLOTS, CH), jnp.int32),
            pltpu.VMEM((SLOTS, CH), jnp.float32),
            pltpu.SemaphoreType.DMA,
            pltpu.SemaphoreType.DMA,
        ],
        compiler_params=pltpu.CompilerParams(needs_layout_passes=False),
    )
    def kfn(xpack, pidx, vals, bias, out, xab, ya, yb, iv, vv, semx, semd):
        wid = lax.axis_index("s") * NC + lax.axis_index("c")

        pltpu.async_copy(xpack.at[wid], xab, semx)
        pltpu.async_copy(bias, ya, semx)
        pltpu.async_copy(bias, yb, semx)

        def start(c, slot):
            pltpu.make_async_copy(
                pidx.at[pl.ds(c * CH, CH)], iv.at[slot], semd).start()
            pltpu.make_async_copy(
                vals.at[pl.ds(c * CH, CH)], vv.at[slot], semd).start()

        def wait(c, slot):
            pltpu.make_async_copy(
                pidx.at[pl.ds(c * CH, CH)], iv.at[slot], semd).wait()
            pltpu.make_async_copy(
                vals.at[pl.ds(c * CH, CH)], vv.at[slot], semd).wait()

        for s in range(SLOTS):
            start(s, s)
        pltpu.make_async_copy(xpack.at[wid], xab, semx).wait()
        pltpu.make_async_copy(bias, ya, semx).wait()
        pltpu.make_async_copy(bias, yb, semx).wait()

        def chunk_body(c, slot):
            wait(c, slot)

            def groups(g, _):
                base = g * (L * GI)
                pvs = [iv[slot, pl.ds(base + k * L, L)] for k in range(GI)]
                vs = [vv[slot, pl.ds(base + k * L, L)] for k in range(GI)]
                col = [pv & 0xFFFF for pv in pvs]
                row = [lax.shift_right_logical(pv, 16) for pv in pvs]
                pk = [plsc.load_gather(xab, [c2]) for c2 in col]
                ab = [
                    plsc.unpack(
                        plsc.bitcast(p, jnp.bfloat16),
                        format=plsc.PackFormat.INTERLEAVED,
                        preferred_element_type=jnp.float32,
                    )
                    for p in pk
                ]
                for k in range(GI):
                    plsc.addupdate_scatter(ya, [row[k]], ab[k][0] * vs[k])
                    plsc.addupdate_scatter(yb, [row[k]], ab[k][1] * vs[k])
                return 0

            lax.fori_loop(0, GRP // GI, groups, 0)

        def outer(i, _):
            for s in range(SLOTS):
                c = i * SLOTS + s
                chunk_body(c, s)
                @pl.when(c + SLOTS < nchunks)
                def _():
                    start(c + SLOTS, s)
            return 0

        lax.fori_loop(0, nchunks // SLOTS, outer, 0)

        pltpu.sync_copy(ya, out.at[wid])
        pltpu.sync_copy(yb, out.at[wid + NW])

    return kfn


def kernel(x, sparse_weight_indices, sparse_weight_values, bias):
    rows = sparse_weight_indices[0]
    cols = sparse_weight_indices[1]
    nnz = rows.shape[0]

    nnzp = ((nnz + SLOTS * CH - 1) // (SLOTS * CH)) * (SLOTS * CH)
    pad = nnzp - nnz
    pidx = (rows.astype(jnp.int32) << 16) | cols.astype(jnp.int32)
    pidx = jnp.pad(pidx, (0, pad))
    vals = jnp.pad(sparse_weight_values, (0, pad))

    lo = lax.bitcast_convert_type(
        x[:NW].astype(jnp.bfloat16), jnp.uint16).astype(jnp.uint32)
    hi = lax.bitcast_convert_type(
        x[NW:].astype(jnp.bfloat16), jnp.uint16).astype(jnp.uint32)
    xpack = lax.bitcast_convert_type((hi << 16) | lo, jnp.int32)

    return _sc_spmm(nnzp // CH)(xpack, pidx, vals, bias)
